# Initial kernel scaffold; baseline (speedup 1.0000x reference)
import jax
import jax.numpy as jnp
from jax.experimental import pallas as pl


def kernel(x_idx, y_idx, semb, temb, M):
    raise NotImplementedError("write your pallas kernel here")


# speedup vs baseline: 2.9213x; 2.9213x over previous
import functools

import jax
import jax.numpy as jnp
from jax import lax
from jax.experimental import pallas as pl
from jax.experimental.pallas import tpu as pltpu
from jax.experimental.pallas import tpu_sc as plsc

V = 100000
D = 128
B = 16384
CHUNK = 128


@functools.lru_cache(maxsize=None)
def _build_gather():
    info = plsc.get_sparse_core_info()
    nc, ns = info.num_cores, info.num_subcores
    nw = nc * ns
    rows_per_w = B // nw
    chunks = rows_per_w // CHUNK

    mesh = plsc.VectorSubcoreMesh(core_axis_name="c", subcore_axis_name="s")

    @functools.partial(
        pl.kernel,
        mesh=mesh,
        out_type=(
            jax.ShapeDtypeStruct((B, D), jnp.float32),
            jax.ShapeDtypeStruct((B, D), jnp.float32),
        ),
        scratch_types=[
            pltpu.VMEM((chunks, CHUNK), jnp.int32),
            pltpu.VMEM((chunks, CHUNK), jnp.int32),
            pltpu.VMEM((CHUNK, D), jnp.float32),
            pltpu.SemaphoreType.DMA,
        ],
    )
    def gather(semb, temb, xi_hbm, yi_hbm, out_x, out_y, xi_v, yi_v, rows_v, sem):
        wid = lax.axis_index("s") * nc + lax.axis_index("c")
        ib = wid * chunks
        pltpu.sync_copy(xi_hbm.at[pl.ds(ib, chunks)], xi_v)
        pltpu.sync_copy(yi_hbm.at[pl.ds(ib, chunks)], yi_v)
        for j in range(chunks):
            pltpu.async_copy(semb.at[xi_v.at[j]], rows_v, sem).wait()
            pltpu.sync_copy(rows_v, out_x.at[pl.ds((ib + j) * CHUNK, CHUNK)])
        for j in range(chunks):
            pltpu.async_copy(temb.at[yi_v.at[j]], rows_v, sem).wait()
            pltpu.sync_copy(rows_v, out_y.at[pl.ds((ib + j) * CHUNK, CHUNK)])

    return gather


def _project(xg, yg, m):
    blk = 2048

    def body(m_ref, x_ref, y_ref, ox_ref, oy_ref):
        mm = m_ref[...]
        dn = (((1,), (1,)), ((), ()))
        ox_ref[...] = lax.dot_general(x_ref[...], mm, dn,
                                      preferred_element_type=jnp.float32)
        oy_ref[...] = lax.dot_general(y_ref[...], mm, dn,
                                      preferred_element_type=jnp.float32)

    return pl.pallas_call(
        body,
        grid=(B // blk,),
        in_specs=[
            pl.BlockSpec((D, D), lambda i: (0, 0)),
            pl.BlockSpec((blk, D), lambda i: (i, 0)),
            pl.BlockSpec((blk, D), lambda i: (i, 0)),
        ],
        out_specs=[
            pl.BlockSpec((blk, D), lambda i: (i, 0)),
            pl.BlockSpec((blk, D), lambda i: (i, 0)),
        ],
        out_shape=[jax.ShapeDtypeStruct((B, D), jnp.float32)] * 2,
    )(m, xg, yg)


def kernel(x_idx, y_idx, semb, temb, M):
    xi = x_idx.astype(jnp.int32).reshape(B // CHUNK, CHUNK)
    yi = y_idx.astype(jnp.int32).reshape(B // CHUNK, CHUNK)
    xg, yg = _build_gather()(semb, temb, xi, yi)
    return _project(xg, yg, M)


# speedup vs baseline: 3.1988x; 1.0950x over previous
import functools

import jax
import jax.numpy as jnp
from jax import lax
from jax.experimental import pallas as pl
from jax.experimental.pallas import tpu as pltpu
from jax.experimental.pallas import tpu_sc as plsc

V = 100000
D = 128
B = 16384
CHUNK = 128


@functools.lru_cache(maxsize=None)
def _build_gather():
    info = plsc.get_sparse_core_info()
    nc, ns = info.num_cores, info.num_subcores
    nw = nc * ns
    rows_per_w = B // nw
    chunks = rows_per_w // CHUNK

    mesh = plsc.VectorSubcoreMesh(core_axis_name="c", subcore_axis_name="s")

    @functools.partial(
        pl.kernel,
        mesh=mesh,
        out_type=(
            jax.ShapeDtypeStruct((B, D), jnp.float32),
            jax.ShapeDtypeStruct((B, D), jnp.float32),
        ),
        scratch_types=[
            pltpu.VMEM((chunks, CHUNK), jnp.int32),
            pltpu.VMEM((chunks, CHUNK), jnp.int32),
            pltpu.VMEM((CHUNK, D), jnp.float32),
            pltpu.VMEM((CHUNK, D), jnp.float32),
            pltpu.SemaphoreType.DMA,
            pltpu.SemaphoreType.DMA,
        ],
    )
    def gather(semb, temb, xi_hbm, yi_hbm, out_x, out_y,
               xi_v, yi_v, rows0, rows1, sem0, sem1):
        wid = lax.axis_index("s") * nc + lax.axis_index("c")
        ib = wid * chunks
        pltpu.sync_copy(xi_hbm.at[pl.ds(ib, chunks)], xi_v)
        pltpu.sync_copy(yi_hbm.at[pl.ds(ib, chunks)], yi_v)
        tasks = ([(semb, xi_v, out_x, j) for j in range(chunks)]
                 + [(temb, yi_v, out_y, j) for j in range(chunks)])
        bufs, sems = (rows0, rows1), (sem0, sem1)
        copies = {}
        tbl0, iv0, _, j0 = tasks[0]
        copies[0] = pltpu.async_copy(tbl0.at[iv0.at[j0]], bufs[0], sems[0])
        for i, (tbl, iv, out, j) in enumerate(tasks):
            if i + 1 < len(tasks):
                ntbl, niv, _, nj = tasks[i + 1]
                copies[(i + 1) % 2] = pltpu.async_copy(
                    ntbl.at[niv.at[nj]], bufs[(i + 1) % 2], sems[(i + 1) % 2])
            copies[i % 2].wait()
            pltpu.sync_copy(bufs[i % 2], out.at[pl.ds((ib + j) * CHUNK, CHUNK)])

    return gather


def _project(xg, yg, m):
    blk = 2048

    def body(m_ref, x_ref, y_ref, ox_ref, oy_ref):
        mm = m_ref[...]
        dn = (((1,), (1,)), ((), ()))
        ox_ref[...] = lax.dot_general(x_ref[...], mm, dn,
                                      preferred_element_type=jnp.float32)
        oy_ref[...] = lax.dot_general(y_ref[...], mm, dn,
                                      preferred_element_type=jnp.float32)

    return pl.pallas_call(
        body,
        grid=(B // blk,),
        in_specs=[
            pl.BlockSpec((D, D), lambda i: (0, 0)),
            pl.BlockSpec((blk, D), lambda i: (i, 0)),
            pl.BlockSpec((blk, D), lambda i: (i, 0)),
        ],
        out_specs=[
            pl.BlockSpec((blk, D), lambda i: (i, 0)),
            pl.BlockSpec((blk, D), lambda i: (i, 0)),
        ],
        out_shape=[jax.ShapeDtypeStruct((B, D), jnp.float32)] * 2,
    )(m, xg, yg)


def kernel(x_idx, y_idx, semb, temb, M):
    xi = x_idx.astype(jnp.int32).reshape(B // CHUNK, CHUNK)
    yi = y_idx.astype(jnp.int32).reshape(B // CHUNK, CHUNK)
    xg, yg = _build_gather()(semb, temb, xi, yi)
    return _project(xg, yg, M)
